# baseline (speedup 1.0000x reference)
import jax
import jax.numpy as jnp
from jax.experimental import pallas as pl
from jax.experimental.pallas import tpu as pltpu

_ROWS = 8


def _body(t_ref, sac_ref, somac_ref, x_ref, n_ref, o_ref):
    i = pl.program_id(0)
    for r in range(_ROWS):
        tb = t_ref[i * _ROWS + r]
        c1 = sac_ref[tb]
        c2 = somac_ref[tb]
        o_ref[r, :] = c1 * x_ref[r, :] + c2 * n_ref[r, :]


def kernel(data_start, t, noise, sqrt_alphas_cumprod, sqrt_one_minus_alphas_cumprod):
    B, N, D = data_start.shape
    C = N * D
    x2 = data_start.reshape(B, C)
    n2 = noise.reshape(B, C)

    grid_spec = pltpu.PrefetchScalarGridSpec(
        num_scalar_prefetch=3,
        grid=(B // _ROWS,),
        in_specs=[
            pl.BlockSpec((_ROWS, C), lambda i, *_: (i, 0)),
            pl.BlockSpec((_ROWS, C), lambda i, *_: (i, 0)),
        ],
        out_specs=pl.BlockSpec((_ROWS, C), lambda i, *_: (i, 0)),
    )

    out = pl.pallas_call(
        _body,
        grid_spec=grid_spec,
        out_shape=jax.ShapeDtypeStruct((B, C), jnp.float32),
    )(t, sqrt_alphas_cumprod, sqrt_one_minus_alphas_cumprod, x2, n2)
    return out.reshape(B, N, D)


# speedup vs baseline: 8.1402x; 8.1402x over previous
import jax
import jax.numpy as jnp
from jax.experimental import pallas as pl
from jax.experimental.pallas import tpu as pltpu

_ROWS = 8


def _body(t_ref, sac_ref, somac_ref, x_ref, n_ref, o_ref):
    i = pl.program_id(0)
    for r in range(_ROWS):
        tb = t_ref[i * _ROWS + r]
        c1 = sac_ref[tb]
        c2 = somac_ref[tb]
        o_ref[r, :, :] = c1 * x_ref[r, :, :] + c2 * n_ref[r, :, :]


def kernel(data_start, t, noise, sqrt_alphas_cumprod, sqrt_one_minus_alphas_cumprod):
    B, N, D = data_start.shape
    xt = jnp.transpose(data_start, (0, 2, 1))
    nt = jnp.transpose(noise, (0, 2, 1))

    grid_spec = pltpu.PrefetchScalarGridSpec(
        num_scalar_prefetch=3,
        grid=(B // _ROWS,),
        in_specs=[
            pl.BlockSpec((_ROWS, D, N), lambda i, *_: (i, 0, 0)),
            pl.BlockSpec((_ROWS, D, N), lambda i, *_: (i, 0, 0)),
        ],
        out_specs=pl.BlockSpec((_ROWS, D, N), lambda i, *_: (i, 0, 0)),
    )

    out_t = pl.pallas_call(
        _body,
        grid_spec=grid_spec,
        out_shape=jax.ShapeDtypeStruct((B, D, N), jnp.float32),
    )(t, sqrt_alphas_cumprod, sqrt_one_minus_alphas_cumprod, xt, nt)
    return jnp.transpose(out_t, (0, 2, 1))
